# baseline (speedup 1.0000x reference)
import functools

import jax
import jax.numpy as jnp
import numpy as np
from jax import lax
from jax.experimental import pallas as pl
from jax.experimental.pallas import tpu as pltpu
from jax.experimental.pallas import tpu_sc as plsc

_GAMMA = 12.0
_L = 16
_NC = 2
_NS = 16
_NW = _NC * _NS
_B = 16384
_D = 64
_V = 1000000
_WENT = 1024

_NPOS_N = 2 * _B
_NPOS_R = _B
_PN = _NPOS_N // _NW
_PR = _NPOS_R // _NW
_WSH = 7
_WIN = 1 << _WSH
_RING = 8


def _emit_positions(tab3, keys_v, blk8_v, out_v, win_s, c_flat, n_groups,
                    pos_base_flat, sem):
    iota = lax.iota(jnp.int32, _L)
    ft = [(q * _L + iota) >> 3 for q in range(4)]
    fi = [(q * _L + iota) & 7 for q in range(4)]

    def drain_one():
        pltpu.make_async_copy(
            tab3.at[:, :, pl.ds(0, _WIN)], blk8_v.at[0], sem).wait()

    def prefetch(col_scalar, slot_scalar):
        off = pl.multiple_of(col_scalar * _WIN, _WIN)
        pltpu.async_copy(tab3.at[:, :, pl.ds(off, _WIN)],
                         blk8_v.at[slot_scalar], sem)

    k0 = plsc.load_gather(
        keys_v, [jnp.broadcast_to(jnp.int32(0), (_L,)),
                 jnp.broadcast_to(jnp.int32(0), (_L,))])[0]
    c0 = lax.shift_right_logical(k0, _WSH)
    for j in range(_RING - 1):
        prefetch(c0 + j, j)

    win_s[0] = c0 - 1
    win_s[1] = jnp.int32(_RING - 1)
    win_s[2] = c0 + (_RING - 1)

    def advance(state, c2):
        def cond(s):
            return s[0] < c2

        def body(s):
            cur, slot, nxt = s
            drain_one()
            prefetch(nxt, slot)
            return (cur + 1, (slot + 1) % _RING, nxt + 1)

        return lax.while_loop(cond, body, state)

    def superblock(sb, carry):
        state = (win_s[0], win_s[1], win_s[2])
        for sg in range(4):
            g = sb * 4 + sg
            row = g // 8
            col = (g % 8) * _L
            kvec = plsc.load_gather(
                keys_v, [jnp.broadcast_to(row, (_L,)), col + iota])
            for k in range(_L):
                key = kvec[k]
                state = advance(state, lax.shift_right_logical(key, _WSH))
                slot = jnp.broadcast_to(state[1], (_L,))
                le = jnp.broadcast_to(jnp.bitwise_and(key, _WIN - 1), (_L,))
                for q in range(4):
                    vals = plsc.load_gather(blk8_v, [slot, ft[q], fi[q], le])
                    out_v[pl.ds(sg * 1024 + k * 64 + q * _L, _L)] = vals
        win_s[0], win_s[1], win_s[2] = state
        pltpu.sync_copy(
            out_v, c_flat.at[pl.ds(pos_base_flat + sb * 4096, 4096)])
        return carry

    lax.fori_loop(0, n_groups // 4, superblock, 0)
    for _ in range(_RING - 1):
        drain_one()


def _phase_a(tab3n, tab3r, nk, rk, cn_flat, cr_flat,
             keys_v, blk8_v, out_v, win_s, sem):
    wid = lax.axis_index("s") * _NC + lax.axis_index("c")

    pltpu.sync_copy(nk.at[wid], keys_v)
    _emit_positions(tab3n, keys_v, blk8_v, out_v, win_s, cn_flat,
                    _PN // _L, wid * _PN * _D, sem)

    pltpu.sync_copy(rk.at[wid], keys_v)
    _emit_positions(tab3r, keys_v, blk8_v, out_v, win_s, cr_flat,
                    _PR // _L, wid * _PR * _D, sem)


def _phase_b(cn2, cr2, rowh, rowt, rowr, out_hbm,
             ih_v, it_v, ir_v,
             ha_v, hb_v, ta_v, tb_v, ra_v, rb_v, out_v, sem):
    wid = lax.axis_index("s") * _NC + lax.axis_index("c")
    pltpu.sync_copy(rowh.at[wid], ih_v)
    pltpu.sync_copy(rowt.at[wid], it_v)
    pltpu.sync_copy(rowr.at[wid], ir_v)

    hbufs = [ha_v, hb_v]
    tbufs = [ta_v, tb_v]
    rbufs = [ra_v, rb_v]

    def fire(c, slot):
        return [pltpu.async_copy(cn2.at[ih_v.at[c]], hbufs[slot], sem),
                pltpu.async_copy(cn2.at[it_v.at[c]], tbufs[slot], sem),
                pltpu.async_copy(cr2.at[ir_v.at[c]], rbufs[slot], sem)]

    iota = lax.iota(jnp.int32, _L)
    descs = fire(0, 0)
    for c in range(4):
        slot = c & 1
        descs_next = fire(c + 1, slot ^ 1) if c < 3 else []
        for d_ in descs:
            d_.wait()
        descs, cur_descs = descs_next, descs

        def group(g, carry):
            rows = g * _L + iota
            accs = [jnp.zeros((_L,), jnp.float32) for _ in range(4)]
            for d in range(_D):
                dv = jnp.broadcast_to(jnp.int32(d), (_L,))
                hv = plsc.load_gather(hbufs[slot], [rows, dv])
                tv = plsc.load_gather(tbufs[slot], [rows, dv])
                rv = plsc.load_gather(rbufs[slot], [rows, dv])
                accs[d & 3] = accs[d & 3] + jnp.abs(hv + rv - tv)
            acc = (accs[0] + accs[1]) + (accs[2] + accs[3])
            out_v[pl.ds(c * 128 + g * _L, _L)] = _GAMMA - acc
            return carry

        lax.fori_loop(0, 128 // _L, group, 0)
    pltpu.sync_copy(out_v, out_hbm.at[pl.ds(wid * (_B // _NW), _B // _NW)])


@jax.jit
def kernel(node_embeddings, rel_embeddings, triplets):
    idx = triplets.astype(jnp.int32)
    nk = jnp.concatenate([idx[:, 0], idx[:, 2]])
    rk = idx[:, 1]
    it32 = lax.iota(jnp.int32, _NPOS_N)
    it16 = lax.iota(jnp.int32, _NPOS_R)
    nk_s, nperm = lax.sort([nk, it32], num_keys=1)
    _, inv_n = lax.sort([nperm, it32], num_keys=1)
    rk_s, rperm = lax.sort([rk, it16], num_keys=1)
    _, inv_r = lax.sort([rperm, it16], num_keys=1)

    nkA = nk_s.reshape(_NW, 8, 128)
    rkA = jnp.concatenate(
        [rk_s.reshape(_NW, _PR), jnp.zeros((_NW, _PR), jnp.int32)],
        axis=1).reshape(_NW, 8, 128)

    ph, pt, pr = inv_n[:_B], inv_n[_B:], inv_r
    rowh = ph.reshape(_NW, 4, 128)
    rowt = pt.reshape(_NW, 4, 128)
    rowr = pr.reshape(_NW, 4, 128)

    tab3n = node_embeddings.T.reshape(8, 8, _V)
    tab3r = rel_embeddings.T.reshape(8, 8, _V)

    mesh = plsc.VectorSubcoreMesh(core_axis_name="c", subcore_axis_name="s")

    run_a = functools.partial(
        pl.kernel,
        mesh=mesh,
        out_type=(jax.ShapeDtypeStruct((_NPOS_N * _D,), jnp.float32),
                  jax.ShapeDtypeStruct((_NPOS_R * _D,), jnp.float32)),
        compiler_params=pltpu.CompilerParams(
            needs_layout_passes=False, use_tc_tiling_on_sc=True,
            disable_bounds_checks=True),
        scratch_types=[
            pltpu.VMEM((8, 128), jnp.int32),
            pltpu.VMEM((_RING, 8, 8, _WIN), jnp.float32),
            pltpu.VMEM((4096,), jnp.float32),
            pltpu.SMEM((4,), jnp.int32),
            pltpu.SemaphoreType.DMA,
        ],
    )(_phase_a)
    cn_flat, cr_flat = run_a(tab3n, tab3r, nkA, rkA)

    cn2 = cn_flat.reshape(_NPOS_N, _D)
    cr2 = cr_flat.reshape(_NPOS_R, _D)

    run_b = functools.partial(
        pl.kernel,
        mesh=mesh,
        out_type=jax.ShapeDtypeStruct((_B,), jnp.float32),
        compiler_params=pltpu.CompilerParams(
            needs_layout_passes=False, use_tc_tiling_on_sc=False,
            disable_bounds_checks=True),
        scratch_types=[
            pltpu.VMEM((4, 128), jnp.int32),
            pltpu.VMEM((4, 128), jnp.int32),
            pltpu.VMEM((4, 128), jnp.int32),
            pltpu.VMEM((128, _D), jnp.float32),
            pltpu.VMEM((128, _D), jnp.float32),
            pltpu.VMEM((128, _D), jnp.float32),
            pltpu.VMEM((128, _D), jnp.float32),
            pltpu.VMEM((128, _D), jnp.float32),
            pltpu.VMEM((128, _D), jnp.float32),
            pltpu.VMEM((_B // _NW,), jnp.float32),
            pltpu.SemaphoreType.DMA,
        ],
    )(_phase_b)
    return run_b(cn2, cr2, rowh, rowt, rowr)


# speedup vs baseline: 1.0575x; 1.0575x over previous
import functools

import jax
import jax.numpy as jnp
from jax import lax
from jax.experimental import pallas as pl
from jax.experimental.pallas import tpu as pltpu
from jax.experimental.pallas import tpu_sc as plsc

_GAMMA = 12.0
_L = 16
_NC = 2
_NS = 16
_NW = _NC * _NS
_B = 16384
_D = 64
_V = 1000000

_NPOS_N = 2 * _B
_NPOS_R = _B
_PN = _NPOS_N // _NW
_PR = _NPOS_R // _NW
_WSH = 7
_WIN = 1 << _WSH
_RING = 8


def _emit_positions(tab3, keys_v, blk8_v, out_v, win_s, c_flat, n_groups,
                    pos_base_flat, sem):
    iota = lax.iota(jnp.int32, _L)
    ft = [(q * _L + iota) >> 3 for q in range(4)]
    fi = [(q * _L + iota) & 7 for q in range(4)]

    def drain_one():
        pltpu.make_async_copy(
            tab3.at[:, :, pl.ds(0, _WIN)], blk8_v.at[0], sem).wait()

    def prefetch(col_scalar, slot_scalar):
        off = pl.multiple_of(col_scalar * _WIN, _WIN)
        pltpu.async_copy(tab3.at[:, :, pl.ds(off, _WIN)],
                         blk8_v.at[slot_scalar], sem)

    k0 = plsc.load_gather(
        keys_v, [jnp.broadcast_to(jnp.int32(0), (_L,)),
                 jnp.broadcast_to(jnp.int32(0), (_L,))])[0]
    c0 = lax.shift_right_logical(k0, _WSH)
    for j in range(_RING - 1):
        prefetch(c0 + j, j)

    win_s[0] = c0 - 1
    win_s[1] = jnp.int32(_RING - 1)
    win_s[2] = c0 + (_RING - 1)

    def advance(state, c2):
        def cond(s):
            return s[0] < c2

        def body(s):
            cur, slot, nxt = s
            drain_one()
            prefetch(nxt, slot)
            return (cur + 1, (slot + 1) % _RING, nxt + 1)

        return lax.while_loop(cond, body, state)

    def superblock(sb, carry):
        state = (win_s[0], win_s[1], win_s[2])
        for sg in range(4):
            g = sb * 4 + sg
            row = g // 8
            col = (g % 8) * _L
            kvec = plsc.load_gather(
                keys_v, [jnp.broadcast_to(row, (_L,)), col + iota])
            for k in range(_L):
                key = kvec[k]
                state = advance(state, lax.shift_right_logical(key, _WSH))
                slot = jnp.broadcast_to(state[1], (_L,))
                le = jnp.broadcast_to(jnp.bitwise_and(key, _WIN - 1), (_L,))
                for q in range(4):
                    vals = plsc.load_gather(blk8_v, [slot, ft[q], fi[q], le])
                    out_v[pl.ds(sg * 1024 + k * 64 + q * _L, _L)] = vals
        win_s[0], win_s[1], win_s[2] = state
        pltpu.sync_copy(
            out_v, c_flat.at[pl.ds(pos_base_flat + sb * 4096, 4096)])
        return carry

    lax.fori_loop(0, n_groups // 4, superblock, 0)
    for _ in range(_RING - 1):
        drain_one()


def _phase_a(tab3n, tab3r, nk, rk, cn_flat, cr_flat,
             keys_v, blk8_v, out_v, win_s, sem):
    wid = lax.axis_index("s") * _NC + lax.axis_index("c")

    pltpu.sync_copy(nk.at[wid], keys_v)
    _emit_positions(tab3n, keys_v, blk8_v, out_v, win_s, cn_flat,
                    _PN // _L, wid * _PN * _D, sem)

    pltpu.sync_copy(rk.at[wid], keys_v)
    _emit_positions(tab3r, keys_v, blk8_v, out_v, win_s, cr_flat,
                    _PR // _L, wid * _PR * _D, sem)


def _phase_b(cn2, cr2, rowh, colh, rowt, colt, rowr, colr, out_hbm,
             ih_v, ch_v, it_v, ct_v, ir_v, cr_v,
             ha_v, hb_v, ta_v, tb_v, ra_v, rb_v, out_v, sem):
    wid = lax.axis_index("s") * _NC + lax.axis_index("c")
    pltpu.sync_copy(rowh.at[wid], ih_v)
    pltpu.sync_copy(colh.at[wid], ch_v)
    pltpu.sync_copy(rowt.at[wid], it_v)
    pltpu.sync_copy(colt.at[wid], ct_v)
    pltpu.sync_copy(rowr.at[wid], ir_v)
    pltpu.sync_copy(colr.at[wid], cr_v)

    hbufs = [ha_v, hb_v]
    tbufs = [ta_v, tb_v]
    rbufs = [ra_v, rb_v]

    def fire(c, slot):
        return [pltpu.async_copy(cn2.at[ih_v.at[c]], hbufs[slot], sem),
                pltpu.async_copy(cn2.at[it_v.at[c]], tbufs[slot], sem),
                pltpu.async_copy(cr2.at[ir_v.at[c]], rbufs[slot], sem)]

    iota = lax.iota(jnp.int32, _L)
    descs = fire(0, 0)
    for c in range(4):
        slot = c & 1
        descs_next = fire(c + 1, slot ^ 1) if c < 3 else []
        for d_ in descs:
            d_.wait()
        descs = descs_next

        def group(g, carry):
            rows = g * _L + iota
            cbh = plsc.load_gather(ch_v, [jnp.broadcast_to(c, (_L,)), rows])
            cbt = plsc.load_gather(ct_v, [jnp.broadcast_to(c, (_L,)), rows])
            cbr = plsc.load_gather(cr_v, [jnp.broadcast_to(c, (_L,)), rows])
            accs = [jnp.zeros((_L,), jnp.float32) for _ in range(4)]
            for d in range(_D):
                hv = plsc.load_gather(hbufs[slot], [rows, cbh + d])
                tv = plsc.load_gather(tbufs[slot], [rows, cbt + d])
                rv = plsc.load_gather(rbufs[slot], [rows, cbr + d])
                accs[d & 3] = accs[d & 3] + jnp.abs(hv + rv - tv)
            acc = (accs[0] + accs[1]) + (accs[2] + accs[3])
            out_v[pl.ds(c * 128 + g * _L, _L)] = _GAMMA - acc
            return carry

        lax.fori_loop(0, 128 // _L, group, 0)
    pltpu.sync_copy(out_v, out_hbm.at[pl.ds(wid * (_B // _NW), _B // _NW)])


@jax.jit
def kernel(node_embeddings, rel_embeddings, triplets):
    idx = triplets.astype(jnp.int32)
    nk = jnp.concatenate([idx[:, 0], idx[:, 2]])
    rk = idx[:, 1]
    it32 = lax.iota(jnp.int32, _NPOS_N)
    it16 = lax.iota(jnp.int32, _NPOS_R)
    nk_s, nperm = lax.sort([nk, it32], num_keys=1)
    _, inv_n = lax.sort([nperm, it32], num_keys=1)
    rk_s, rperm = lax.sort([rk, it16], num_keys=1)
    _, inv_r = lax.sort([rperm, it16], num_keys=1)

    nkA = nk_s.reshape(_NW, 8, 128)
    rkA = jnp.concatenate(
        [rk_s.reshape(_NW, _PR), jnp.zeros((_NW, _PR), jnp.int32)],
        axis=1).reshape(_NW, 8, 128)

    ph, pt, pr = inv_n[:_B], inv_n[_B:], inv_r
    rowh = (ph >> 1).reshape(_NW, 4, 128)
    colh = ((ph & 1) * _D).reshape(_NW, 4, 128)
    rowt = (pt >> 1).reshape(_NW, 4, 128)
    colt = ((pt & 1) * _D).reshape(_NW, 4, 128)
    rowr = (pr >> 1).reshape(_NW, 4, 128)
    colr = ((pr & 1) * _D).reshape(_NW, 4, 128)

    tab3n = node_embeddings.T.reshape(8, 8, _V)
    tab3r = rel_embeddings.T.reshape(8, 8, _V)

    mesh = plsc.VectorSubcoreMesh(core_axis_name="c", subcore_axis_name="s")

    run_a = functools.partial(
        pl.kernel,
        mesh=mesh,
        out_type=(jax.ShapeDtypeStruct((_NPOS_N * _D,), jnp.float32),
                  jax.ShapeDtypeStruct((_NPOS_R * _D,), jnp.float32)),
        compiler_params=pltpu.CompilerParams(
            needs_layout_passes=False, use_tc_tiling_on_sc=True,
            disable_bounds_checks=True),
        scratch_types=[
            pltpu.VMEM((8, 128), jnp.int32),
            pltpu.VMEM((_RING, 8, 8, _WIN), jnp.float32),
            pltpu.VMEM((4096,), jnp.float32),
            pltpu.SMEM((4,), jnp.int32),
            pltpu.SemaphoreType.DMA,
        ],
    )(_phase_a)
    cn_flat, cr_flat = run_a(tab3n, tab3r, nkA, rkA)

    cn2 = cn_flat.reshape(_NPOS_N // 2, 2 * _D)
    cr2 = cr_flat.reshape(_NPOS_R // 2, 2 * _D)

    run_b = functools.partial(
        pl.kernel,
        mesh=mesh,
        out_type=jax.ShapeDtypeStruct((_B,), jnp.float32),
        compiler_params=pltpu.CompilerParams(
            needs_layout_passes=False, use_tc_tiling_on_sc=False,
            disable_bounds_checks=True),
        scratch_types=[
            pltpu.VMEM((4, 128), jnp.int32),
            pltpu.VMEM((4, 128), jnp.int32),
            pltpu.VMEM((4, 128), jnp.int32),
            pltpu.VMEM((4, 128), jnp.int32),
            pltpu.VMEM((4, 128), jnp.int32),
            pltpu.VMEM((4, 128), jnp.int32),
            pltpu.VMEM((128, 128), jnp.float32),
            pltpu.VMEM((128, 128), jnp.float32),
            pltpu.VMEM((128, 128), jnp.float32),
            pltpu.VMEM((128, 128), jnp.float32),
            pltpu.VMEM((128, 128), jnp.float32),
            pltpu.VMEM((128, 128), jnp.float32),
            pltpu.VMEM((_B // _NW,), jnp.float32),
            pltpu.SemaphoreType.DMA,
        ],
    )(_phase_b)
    return run_b(cn2, cr2, rowh, colh, rowt, colt, rowr, colr)
